# baseline (speedup 1.0000x reference)
import functools

import jax
import jax.numpy as jnp
from jax import lax
from jax.experimental import pallas as pl
from jax.experimental.pallas import tpu as pltpu
from jax.experimental.pallas import tpu_sc as plsc

N, E, D, H, O, G = 10000, 320000, 128, 128, 64, 64

NC, NS = 2, 16
HH = H // NC
EPT = E // NS
CK = 80
NCHUNK = EPT // CK
NBUF = 5
NP_ = 10240
RPT = NP_ // NS

BR = 2000
NB = N // BR


def _sc_agg_build():
    mesh = plsc.VectorSubcoreMesh(
        core_axis_name="c", subcore_axis_name="s",
        num_cores=NC, num_subcores=NS)

    @functools.partial(
        pl.kernel,
        out_type=jax.ShapeDtypeStruct((NC, NP_, HH), jnp.float32),
        mesh=mesh,
        scratch_types=[
            pltpu.VMEM((NCHUNK, CK), jnp.int32),
            pltpu.VMEM((NCHUNK, CK), jnp.int32),
            [pltpu.VMEM((CK, HH), jnp.float32)] * NBUF,
            [pltpu.SemaphoreType.DMA] * NBUF,
            pltpu.VMEM_SHARED((NP_, HH), jnp.float32),
        ],
        compiler_params=pltpu.CompilerParams(use_tc_tiling_on_sc=False),
    )
    def sc_agg(neigh_hbm, row_hbm, col_hbm, zeros_hbm, out_hbm,
               rowv, colv, bufs, gsems, agg_sh):
        cid = lax.axis_index("c")
        sid = lax.axis_index("s")

        pltpu.sync_copy(row_hbm.at[sid], rowv)
        pltpu.sync_copy(col_hbm.at[sid], colv)
        nh = neigh_hbm.at[cid]

        for b in range(NBUF):
            pltpu.async_copy(nh.at[colv.at[b]], bufs[b], gsems[b])

        pltpu.sync_copy(zeros_hbm, agg_sh.at[pl.ds(sid * RPT, RPT)])
        plsc.subcore_barrier()

        def block(t, carry):
            for b in range(NBUF):
                i = t * NBUF + b
                pltpu.make_async_copy(
                    nh.at[colv.at[i]], bufs[b], gsems[b]).wait()
                pltpu.sync_copy(bufs[b], agg_sh.at[rowv.at[i]], add=True)
                pltpu.async_copy(nh.at[colv.at[i + NBUF]], bufs[b], gsems[b])
            return carry
        lax.fori_loop(0, NCHUNK // NBUF - 1, block, 0)
        for b in range(NBUF):
            i = NCHUNK - NBUF + b
            pltpu.make_async_copy(
                nh.at[colv.at[i]], bufs[b], gsems[b]).wait()
            pltpu.sync_copy(bufs[b], agg_sh.at[rowv.at[i]], add=True)
        plsc.subcore_barrier()

        pltpu.sync_copy(agg_sh.at[pl.ds(sid * RPT, RPT)],
                        out_hbm.at[cid, pl.ds(sid * RPT, RPT)])

    return sc_agg


@functools.cache
def _sc_agg_cached():
    return _sc_agg_build()


def _sc_agg(neigh, row3, col3, zeros2d):
    return _sc_agg_cached()(neigh, row3, col3, zeros2d)


def _mm0_body(x_ref, w_ref, o1_ref, o2_ref):
    out = jnp.dot(x_ref[...], w_ref[...], preferred_element_type=jnp.float32)
    o1_ref[...] = out[:, :H]
    o2_ref[...] = jnp.stack([out[:, H:H + HH], out[:, H + HH:]], axis=0)


def _layer_body(g_ref, w_ref, o_ref):
    o_ref[...] = jnp.dot(g_ref[...], w_ref[...],
                         preferred_element_type=jnp.float32)


def _pool_body(g_ref, b_ref, wc_ref, bc_ref, o_ref,
               sums, counts):
    i = pl.program_id(0)

    @pl.when(i == 0)
    def _():
        sums[...] = jnp.zeros_like(sums)
        counts[...] = jnp.zeros_like(counts)

    g = g_ref[...]
    onehot = (b_ref[...] == lax.broadcasted_iota(jnp.int32, (1, G), 1)
              ).astype(jnp.float32)
    dn = (((0,), (0,)), ((), ()))
    sums[...] += lax.dot_general(onehot, g, dn,
                                 preferred_element_type=jnp.float32)
    counts[...] += lax.dot_general(onehot, jnp.ones((BR, H), jnp.float32), dn,
                                   preferred_element_type=jnp.float32)

    @pl.when(i == NB - 1)
    def _():
        pooled = sums[...] / jnp.maximum(counts[...], 1.0)
        o_ref[...] = jnp.dot(pooled, wc_ref[...],
                             preferred_element_type=jnp.float32) + bc_ref[...]


def _row_spec():
    return pl.BlockSpec((BR, H), lambda i: (i, 0))


def _parts_spec():
    return pl.BlockSpec((NC, BR, HH), lambda i: (0, i, 0))


def _nb2_spec():
    return pl.BlockSpec((2, BR, HH), lambda i: (0, i, 0))


def _mm0(x, w12):
    return pl.pallas_call(
        _mm0_body,
        grid=(NB,),
        in_specs=[_row_spec(), pl.BlockSpec((D, 2 * H), lambda i: (0, 0))],
        out_specs=[_row_spec(), _nb2_spec()],
        out_shape=[jax.ShapeDtypeStruct((N, H), jnp.float32),
                   jax.ShapeDtypeStruct((2, N, HH), jnp.float32)],
    )(x, w12)


def _layer(g, w12):
    return pl.pallas_call(
        _layer_body,
        grid=(NB,),
        in_specs=[_row_spec(), pl.BlockSpec((H, 2 * H), lambda i: (0, 0))],
        out_specs=pl.BlockSpec((BR, 2 * H), lambda i: (i, 0)),
        out_shape=jax.ShapeDtypeStruct((N, 2 * H), jnp.float32),
    )(g, w12)


def _pool(g, batch2, wc_t, bc2):
    return pl.pallas_call(
        _pool_body,
        grid=(NB,),
        in_specs=[_row_spec(),
                  pl.BlockSpec((BR, 1), lambda i: (i, 0)),
                  pl.BlockSpec((H, O), lambda i: (0, 0)),
                  pl.BlockSpec((1, O), lambda i: (0, 0))],
        out_specs=pl.BlockSpec((G, O), lambda i: (0, 0)),
        out_shape=jax.ShapeDtypeStruct((G, O), jnp.float32),
        scratch_shapes=[pltpu.VMEM((G, H), jnp.float32),
                        pltpu.VMEM((G, H), jnp.float32)],
        compiler_params=pltpu.CompilerParams(
            dimension_semantics=("arbitrary",)),
    )(g, batch2, wc_t, bc2)


def kernel(x, edge_index, batch, W1_0, W2_0, W1_1, W2_1, W1_2, W2_2, Wc, bc):
    w0 = jnp.concatenate([W1_0.T, W2_0.T], axis=1)
    w1 = jnp.concatenate([W1_1.T, W2_1.T], axis=1)
    w2 = jnp.concatenate([W1_2.T, W2_2.T], axis=1)
    row3 = edge_index[0].reshape(NS, NCHUNK, CK)
    col3 = edge_index[1].reshape(NS, NCHUNK, CK)
    batch2 = batch.reshape(N, 1)
    bc2 = bc.reshape(1, O)
    zeros2d = jnp.zeros((RPT, HH), jnp.float32)

    def boundary(out):
        nb2 = jnp.stack([out[:, H:H + HH], out[:, H + HH:]], axis=0)
        parts = _sc_agg(nb2, row3, col3, zeros2d)
        agg = jnp.concatenate([parts[0, :N, :], parts[1, :N, :]], axis=1)
        return jnp.maximum(out[:, :H] + agg, 0.0)

    s, nb2 = _mm0(x, w0)
    parts = _sc_agg(nb2, row3, col3, zeros2d)
    agg = jnp.concatenate([parts[0, :N, :], parts[1, :N, :]], axis=1)
    g = jnp.maximum(s + agg, 0.0)
    g = boundary(_layer(g, w1))
    g = boundary(_layer(g, w2))
    return _pool(g, batch2, Wc.T, bc2)


# speedup vs baseline: 1.0896x; 1.0896x over previous
import functools

import jax
import jax.numpy as jnp
from jax import lax
from jax.experimental import pallas as pl
from jax.experimental.pallas import tpu as pltpu
from jax.experimental.pallas import tpu_sc as plsc

N, E, D, H, O, G = 10000, 320000, 128, 128, 64, 64

NC, NS = 2, 16
HH = H // NC
EPT = E // NS
CK = 80
NCHUNK = EPT // CK
NBUF = 5
NP_ = 10240
RPT = NP_ // NS

BR = 2000
NB = N // BR


def _sc_agg_build():
    mesh = plsc.VectorSubcoreMesh(
        core_axis_name="c", subcore_axis_name="s",
        num_cores=NC, num_subcores=NS)

    @functools.partial(
        pl.kernel,
        out_type=jax.ShapeDtypeStruct((NC, NP_, HH), jnp.float32),
        mesh=mesh,
        scratch_types=[
            pltpu.VMEM((NCHUNK, CK), jnp.int32),
            pltpu.VMEM((NCHUNK, CK), jnp.int32),
            [pltpu.VMEM((CK, HH), jnp.float32)] * NBUF,
            [pltpu.SemaphoreType.DMA] * NBUF,
            pltpu.VMEM_SHARED((NP_, HH), jnp.float32),
        ],
        compiler_params=pltpu.CompilerParams(use_tc_tiling_on_sc=False),
    )
    def sc_agg(neigh_hbm, row_hbm, col_hbm, zeros_hbm, out_hbm,
               rowv, colv, bufs, gsems, agg_sh):
        cid = lax.axis_index("c")
        sid = lax.axis_index("s")

        pltpu.sync_copy(row_hbm.at[sid], rowv)
        pltpu.sync_copy(col_hbm.at[sid], colv)
        nh = neigh_hbm.at[cid]

        for b in range(NBUF):
            pltpu.async_copy(nh.at[colv.at[b]], bufs[b], gsems[b])

        pltpu.sync_copy(zeros_hbm, agg_sh.at[pl.ds(sid * RPT, RPT)])
        plsc.subcore_barrier()

        def block(t, carry):
            for b in range(NBUF):
                i = t * NBUF + b
                pltpu.make_async_copy(
                    nh.at[colv.at[i]], bufs[b], gsems[b]).wait()
                pltpu.sync_copy(bufs[b], agg_sh.at[rowv.at[i]], add=True)
                pltpu.async_copy(nh.at[colv.at[i + NBUF]], bufs[b], gsems[b])
            return carry
        lax.fori_loop(0, NCHUNK // NBUF - 1, block, 0)
        for b in range(NBUF):
            i = NCHUNK - NBUF + b
            pltpu.make_async_copy(
                nh.at[colv.at[i]], bufs[b], gsems[b]).wait()
            pltpu.sync_copy(bufs[b], agg_sh.at[rowv.at[i]], add=True)
        plsc.subcore_barrier()

        pltpu.sync_copy(agg_sh.at[pl.ds(sid * RPT, RPT)],
                        out_hbm.at[cid, pl.ds(sid * RPT, RPT)])

    return sc_agg


@functools.cache
def _sc_agg_cached():
    return _sc_agg_build()


def _sc_agg(neigh, row3, col3, zeros2d):
    return _sc_agg_cached()(neigh, row3, col3, zeros2d)


def _mm0_body(x_ref, w_ref, o1_ref, o2_ref):
    out = jnp.dot(x_ref[...], w_ref[...], preferred_element_type=jnp.float32)
    o1_ref[...] = out[:, :H]
    o2_ref[...] = jnp.stack([out[:, H:H + HH], out[:, H + HH:]], axis=0)


def _layer_body(s_ref, p_ref, w_ref, o1_ref, o2_ref):
    agg = jnp.concatenate([p_ref[0], p_ref[1]], axis=1)
    g = jnp.maximum(s_ref[...] + agg, 0.0)
    out = jnp.dot(g, w_ref[...], preferred_element_type=jnp.float32)
    o1_ref[...] = out[:, :H]
    o2_ref[...] = jnp.stack([out[:, H:H + HH], out[:, H + HH:]], axis=0)


def _pool_body(s_ref, p_ref, b_ref, wc_ref, bc_ref, o_ref,
               sums, counts):
    i = pl.program_id(0)

    @pl.when(i == 0)
    def _():
        sums[...] = jnp.zeros_like(sums)
        counts[...] = jnp.zeros_like(counts)

    agg = jnp.concatenate([p_ref[0], p_ref[1]], axis=1)
    g = jnp.maximum(s_ref[...] + agg, 0.0)
    onehot = (b_ref[...] == lax.broadcasted_iota(jnp.int32, (1, G), 1)
              ).astype(jnp.float32)
    dn = (((0,), (0,)), ((), ()))
    sums[...] += lax.dot_general(onehot, g, dn,
                                 preferred_element_type=jnp.float32)
    counts[...] += lax.dot_general(onehot, jnp.ones((BR, H), jnp.float32), dn,
                                   preferred_element_type=jnp.float32)

    @pl.when(i == NB - 1)
    def _():
        pooled = sums[...] / jnp.maximum(counts[...], 1.0)
        o_ref[...] = jnp.dot(pooled, wc_ref[...],
                             preferred_element_type=jnp.float32) + bc_ref[...]


def _row_spec():
    return pl.BlockSpec((BR, H), lambda i: (i, 0))


def _parts_spec():
    return pl.BlockSpec((NC, BR, HH), lambda i: (0, i, 0))


def _nb2_spec():
    return pl.BlockSpec((2, BR, HH), lambda i: (0, i, 0))


def _mm0(x, w12):
    return pl.pallas_call(
        _mm0_body,
        grid=(NB,),
        in_specs=[_row_spec(), pl.BlockSpec((D, 2 * H), lambda i: (0, 0))],
        out_specs=[_row_spec(), _nb2_spec()],
        out_shape=[jax.ShapeDtypeStruct((N, H), jnp.float32),
                   jax.ShapeDtypeStruct((2, N, HH), jnp.float32)],
    )(x, w12)


def _layer(s, parts, w12):
    return pl.pallas_call(
        _layer_body,
        grid=(NB,),
        in_specs=[_row_spec(), _parts_spec(),
                  pl.BlockSpec((H, 2 * H), lambda i: (0, 0))],
        out_specs=[_row_spec(), _nb2_spec()],
        out_shape=[jax.ShapeDtypeStruct((N, H), jnp.float32),
                   jax.ShapeDtypeStruct((2, N, HH), jnp.float32)],
    )(s, parts, w12)


def _pool(s, parts, batch2, wc_t, bc2):
    return pl.pallas_call(
        _pool_body,
        grid=(NB,),
        in_specs=[_row_spec(), _parts_spec(),
                  pl.BlockSpec((BR, 1), lambda i: (i, 0)),
                  pl.BlockSpec((H, O), lambda i: (0, 0)),
                  pl.BlockSpec((1, O), lambda i: (0, 0))],
        out_specs=pl.BlockSpec((G, O), lambda i: (0, 0)),
        out_shape=jax.ShapeDtypeStruct((G, O), jnp.float32),
        scratch_shapes=[pltpu.VMEM((G, H), jnp.float32),
                        pltpu.VMEM((G, H), jnp.float32)],
        compiler_params=pltpu.CompilerParams(
            dimension_semantics=("arbitrary",)),
    )(s, parts, batch2, wc_t, bc2)


def kernel(x, edge_index, batch, W1_0, W2_0, W1_1, W2_1, W1_2, W2_2, Wc, bc):
    w0 = jnp.concatenate([W1_0.T, W2_0.T], axis=1)
    w1 = jnp.concatenate([W1_1.T, W2_1.T], axis=1)
    w2 = jnp.concatenate([W1_2.T, W2_2.T], axis=1)
    row3 = edge_index[0].reshape(NS, NCHUNK, CK)
    col3 = edge_index[1].reshape(NS, NCHUNK, CK)
    batch2 = batch.reshape(N, 1)
    bc2 = bc.reshape(1, O)
    zeros2d = jnp.zeros((RPT, HH), jnp.float32)

    s, nb2 = _mm0(x, w0)
    parts = _sc_agg(nb2, row3, col3, zeros2d)
    s, nb2 = _layer(s, parts, w1)
    parts = _sc_agg(nb2, row3, col3, zeros2d)
    s, nb2 = _layer(s, parts, w2)
    parts = _sc_agg(nb2, row3, col3, zeros2d)
    return _pool(s, parts, batch2, Wc.T, bc2)
